# baseline (speedup 1.0000x reference)
import jax
import jax.numpy as jnp
from jax.experimental import pallas as pl

N = 10000
H = 128


def _apply(lin, x):
    return x @ lin["w"] + lin["b"]


def _readout_body(h_ref, w0, b0, w1, b1, w2, b2, o_ref):
    h = h_ref[...]
    hg_sum = jnp.sum(h, axis=0, keepdims=True)
    hg_max = jnp.max(h, axis=0, keepdims=True)
    hg_mean = hg_sum / N
    hg = jnp.concatenate([hg_sum, hg_max, hg_mean], axis=1)
    x = jax.nn.relu(hg @ w0[...] + b0[...])
    x = jax.nn.relu(x @ w1[...] + b1[...])
    o_ref[...] = x @ w2[...] + b2[...]


def kernel(h, e, edge_index, params):
    h = params["emb_h"][h]
    e = params["emb_e"][e]
    src = edge_index[0]
    dst = edge_index[1]
    for layer in params["layers"]:
        Ah = _apply(layer["A"], h)
        Bh = _apply(layer["B"], h)
        Dh = _apply(layer["D"], h)
        Eh = _apply(layer["Eh"], h)
        Ce = _apply(layer["C"], e)
        e_hat = Ce + Dh[src] + Eh[dst]
        sigma = jax.nn.sigmoid(e_hat)
        num = jax.ops.segment_sum(sigma * Bh[src], dst, num_segments=N)
        den = jax.ops.segment_sum(sigma, dst, num_segments=N)
        h = h + jax.nn.relu(Ah + num / (den + 1e-6))
        e = e + jax.nn.relu(e_hat)
    m = params["mlp"]
    out = pl.pallas_call(
        _readout_body,
        out_shape=jax.ShapeDtypeStruct((1, 1), jnp.float32),
    )(h, m[0]["w"], m[0]["b"].reshape(1, -1), m[1]["w"], m[1]["b"].reshape(1, -1),
      m[2]["w"], m[2]["b"].reshape(1, -1))
    return out


# speedup vs baseline: 1.7781x; 1.7781x over previous
import jax
import jax.numpy as jnp
from jax import lax
from jax.experimental import pallas as pl
from jax.experimental.pallas import tpu as pltpu
from jax.experimental.pallas import tpu_sc as plsc

N = 10000
E = 320000
H = 128
HF = 64
ATOM_PAD = 32
BOND_PAD = 8

TN = 400
TE = 512

NS = 16
EP = E // NS
C = 32
NCHUNK = EP // C
NP = 10240
NPS = NP // NS
ZR = 64


def _node0_body(hid_ref, emb_ref, w_ref, b_ref, h_ref, ah_ref, db_ref, eh_ref):
    idx = hid_ref[...]
    oh = (idx == lax.broadcasted_iota(jnp.int32, (TN, ATOM_PAD), 1)
          ).astype(jnp.float32)
    h = oh @ emb_ref[...]
    h_ref[...] = h
    p = h @ w_ref[...] + b_ref[...]
    ah_ref[...] = p[:, 0:H]
    db_ref[...] = jnp.stack([p[:, 128:256], p[:, 256:384]])
    eh_ref[...] = jnp.stack(
        [p[:, 384:512],
         jnp.concatenate([p[:, 448:512], p[:, 384:448]], axis=1)])


def _nodeu_body(h_ref, ahp_ref, nd_ref, w_ref, b_ref,
                hn_ref, ah_ref, db_ref, eh_ref):
    nd = nd_ref[...]
    num = jnp.concatenate([nd[0, :, 0:HF], nd[1, :, 0:HF]], axis=1)
    den = jnp.concatenate([nd[0, :, HF:H], nd[1, :, HF:H]], axis=1)
    h = h_ref[...] + jnp.maximum(ahp_ref[...] + num / (den + 1e-6), 0.0)
    hn_ref[...] = h
    p = h @ w_ref[...] + b_ref[...]
    ah_ref[...] = p[:, 0:H]
    db_ref[...] = jnp.stack([p[:, 128:256], p[:, 256:384]])
    eh_ref[...] = jnp.stack(
        [p[:, 384:512],
         jnp.concatenate([p[:, 448:512], p[:, 384:448]], axis=1)])


def _edge0_body(eid_ref, emb_ref, w_ref, b_ref, cee_ref):
    idx = eid_ref[...]
    oh = (idx == lax.broadcasted_iota(jnp.int32, (TE, BOND_PAD), 1)
          ).astype(jnp.float32)
    ee = oh @ emb_ref[...]
    ce = ee @ w_ref[...] + b_ref[...]
    cee_ref[...] = jnp.stack(
        [jnp.concatenate([ce[:, 0:HF], ee[:, 0:HF]], axis=1),
         jnp.concatenate([ce[:, HF:H], ee[:, HF:H]], axis=1)])


def _edgeu_body(eo_ref, w_ref, b_ref, cee_ref):
    eo = eo_ref[...]
    ef = jnp.concatenate([eo[0], eo[1]], axis=1)
    ce = ef @ w_ref[...] + b_ref[...]
    cee_ref[...] = jnp.stack(
        [jnp.concatenate([ce[:, 0:HF], eo[0]], axis=1),
         jnp.concatenate([ce[:, HF:H], eo[1]], axis=1)])


def _readout_body(h_ref, ahp_ref, nd_ref, w0, b0, w1, b1, w2, b2, o_ref):
    nd = nd_ref[...]
    num = jnp.concatenate([nd[0, 0:N, 0:HF], nd[1, 0:N, 0:HF]], axis=1)
    den = jnp.concatenate([nd[0, 0:N, HF:H], nd[1, 0:N, HF:H]], axis=1)
    h = h_ref[...] + jnp.maximum(ahp_ref[...] + num / (den + 1e-6), 0.0)
    s = jnp.sum(h, axis=0, keepdims=True)
    m = jnp.max(h, axis=0, keepdims=True)
    hg = jnp.concatenate([s, m, s / N], axis=1)
    x = jnp.maximum(hg @ w0[...] + b0[...], 0.0)
    x = jnp.maximum(x @ w1[...] + b1[...], 0.0)
    o_ref[...] = x @ w2[...] + b2[...]


def _rep(shape):
    return pl.BlockSpec(shape, lambda i: tuple(0 for _ in shape))


_node_out_specs = [
    pl.BlockSpec((TN, H), lambda i: (i, 0)),
    pl.BlockSpec((TN, H), lambda i: (i, 0)),
    pl.BlockSpec((2, TN, H), lambda i: (0, i, 0)),
    pl.BlockSpec((2, TN, H), lambda i: (0, i, 0)),
]
_node_out_shape = [
    jax.ShapeDtypeStruct((N, H), jnp.float32),
    jax.ShapeDtypeStruct((N, H), jnp.float32),
    jax.ShapeDtypeStruct((2, N, H), jnp.float32),
    jax.ShapeDtypeStruct((2, N, H), jnp.float32),
]

_node0 = pl.pallas_call(
    _node0_body,
    grid=(N // TN,),
    in_specs=[
        pl.BlockSpec((TN, 1), lambda i: (i, 0)),
        _rep((ATOM_PAD, H)),
        _rep((H, 512)),
        _rep((1, 512)),
    ],
    out_specs=_node_out_specs,
    out_shape=_node_out_shape,
)

_nodeu = pl.pallas_call(
    _nodeu_body,
    grid=(N // TN,),
    in_specs=[
        pl.BlockSpec((TN, H), lambda i: (i, 0)),
        pl.BlockSpec((TN, H), lambda i: (i, 0)),
        pl.BlockSpec((2, TN, H), lambda i: (0, i, 0)),
        _rep((H, 512)),
        _rep((1, 512)),
    ],
    out_specs=_node_out_specs,
    out_shape=_node_out_shape,
)

_edge0 = pl.pallas_call(
    _edge0_body,
    grid=(E // TE,),
    in_specs=[
        pl.BlockSpec((TE, 1), lambda i: (i, 0)),
        _rep((BOND_PAD, H)),
        _rep((H, H)),
        _rep((1, H)),
    ],
    out_specs=pl.BlockSpec((2, TE, H), lambda i: (0, i, 0)),
    out_shape=jax.ShapeDtypeStruct((2, E, H), jnp.float32),
)

_edgeu = pl.pallas_call(
    _edgeu_body,
    grid=(E // TE,),
    in_specs=[
        pl.BlockSpec((2, TE, HF), lambda i: (0, i, 0)),
        _rep((H, H)),
        _rep((1, H)),
    ],
    out_specs=pl.BlockSpec((2, TE, H), lambda i: (0, i, 0)),
    out_shape=jax.ShapeDtypeStruct((2, E, H), jnp.float32),
)

_readout = pl.pallas_call(
    _readout_body,
    out_shape=jax.ShapeDtypeStruct((1, 1), jnp.float32),
)


def _sc_body(src_hbm, dst_hbm, cee_hbm, db_hbm, ehh_hbm,
             eout_hbm, nd_hbm,
             acc, src_v, dst_v, sadj_v, dadj_v, cee_v, dbr_v, ehr_v,
             stage_v, eo_v, zero_v, sem1, sem2):
    c = lax.axis_index("c")
    s = lax.axis_index("s")
    cn = c * N

    def zrow(j, _):
        for k in range(H // 16):
            zero_v[j, pl.ds(k * 16, 16)] = jnp.zeros((16,), jnp.float32)
        return _
    lax.fori_loop(0, ZR, zrow, 0)
    for i in range(NPS // ZR):
        pltpu.sync_copy(zero_v, acc.at[pl.ds(s * NPS + i * ZR, ZR)])
    plsc.subcore_barrier()

    def edge(j, _2):
        for k in range(HF // 16):
            sl = pl.ds(k * 16, 16)
            sl2 = pl.ds(HF + k * 16, 16)
            ehat = cee_v[j, sl] + dbr_v[j, sl] + ehr_v[j, sl]
            sig = 1.0 / (1.0 + jnp.exp(-ehat))
            stage_v[j, sl] = sig * dbr_v[j, sl2]
            stage_v[j, sl2] = sig
            eo_v[j, sl] = cee_v[j, sl2] + jnp.maximum(ehat, 0.0)
        return _2

    def chunk(g, carry):
        base = s * EP + g * C
        pltpu.sync_copy(src_hbm.at[pl.ds(base, C)], src_v)
        pltpu.sync_copy(dst_hbm.at[pl.ds(base, C)], dst_v)
        for k in range(C // 16):
            sl = pl.ds(k * 16, 16)
            sadj_v[sl] = src_v[sl] + cn
            dadj_v[sl] = dst_v[sl] + cn
        g1 = pltpu.async_copy(db_hbm.at[sadj_v], dbr_v, sem1)
        g2 = pltpu.async_copy(ehh_hbm.at[dadj_v], ehr_v, sem2)
        ebase = c * E + base
        pltpu.sync_copy(cee_hbm.at[pl.ds(ebase, C)], cee_v)
        g1.wait()
        g2.wait()

        lax.fori_loop(0, C, edge, 0)

        pltpu.sync_copy(eo_v, eout_hbm.at[pl.ds(ebase, C)])
        pltpu.sync_copy(stage_v, acc.at[dst_v], add=True)
        return carry
    lax.fori_loop(0, NCHUNK, chunk, 0)

    plsc.subcore_barrier()
    for i in range(NPS // ZR):
        pltpu.sync_copy(acc.at[pl.ds(s * NPS + i * ZR, ZR)], zero_v)
        pltpu.sync_copy(zero_v, nd_hbm.at[pl.ds(c * NP + s * NPS + i * ZR, ZR)])


_sc_layer = pl.kernel(
    _sc_body,
    mesh=plsc.VectorSubcoreMesh(core_axis_name="c", subcore_axis_name="s"),
    out_type=[
        jax.ShapeDtypeStruct((2 * E, HF), jnp.float32),
        jax.ShapeDtypeStruct((2 * NP, H), jnp.float32),
    ],
    scratch_types=[
        pltpu.VMEM_SHARED((NP, H), jnp.float32),
        pltpu.VMEM((C,), jnp.int32),
        pltpu.VMEM((C,), jnp.int32),
        pltpu.VMEM((C,), jnp.int32),
        pltpu.VMEM((C,), jnp.int32),
        pltpu.VMEM((C, H), jnp.float32),
        pltpu.VMEM((C, H), jnp.float32),
        pltpu.VMEM((C, H), jnp.float32),
        pltpu.VMEM((C, H), jnp.float32),
        pltpu.VMEM((C, HF), jnp.float32),
        pltpu.VMEM((ZR, H), jnp.float32),
        pltpu.SemaphoreType.DMA,
        pltpu.SemaphoreType.DMA,
    ],
)


def _pack_node_w(layer):
    wa, wb = layer["A"]["w"], layer["B"]["w"]
    wd, we = layer["D"]["w"], layer["Eh"]["w"]
    w = jnp.concatenate([wa, wd[:, :HF], wb[:, :HF], wd[:, HF:], wb[:, HF:],
                         we], axis=1)
    ba, bb = layer["A"]["b"], layer["B"]["b"]
    bd, be = layer["D"]["b"], layer["Eh"]["b"]
    b = jnp.concatenate([ba, bd[:HF], bb[:HF], bd[HF:], bb[HF:], be])
    return w, b.reshape(1, 512)


def kernel(h, e, edge_index, params):
    layers = params["layers"]
    emb_h = jnp.concatenate(
        [params["emb_h"], jnp.zeros((ATOM_PAD - params["emb_h"].shape[0], H),
                                    jnp.float32)], axis=0)
    emb_e = jnp.concatenate(
        [params["emb_e"], jnp.zeros((BOND_PAD - params["emb_e"].shape[0], H),
                                    jnp.float32)], axis=0)
    src = edge_index[0]
    dst = edge_index[1]
    hid = h.astype(jnp.int32).reshape(N, 1)
    eid = e.astype(jnp.int32).reshape(E, 1)

    w0, b0 = _pack_node_w(layers[0])
    hcur, ah, db, ehf = _node0(hid, emb_h, w0, b0)
    cee = _edge0(eid, emb_e, layers[0]["C"]["w"],
                 layers[0]["C"]["b"].reshape(1, H))

    nd = None
    for l in range(4):
        eout, nd = _sc_layer(src, dst, cee.reshape(2 * E, H),
                             db.reshape(2 * N, H), ehf.reshape(2 * N, H))
        nd = nd.reshape(2, NP, H)
        if l < 3:
            wl, bl = _pack_node_w(layers[l + 1])
            hcur, ah, db, ehf = _nodeu(hcur, ah, nd, wl, bl)
            cee = _edgeu(eout.reshape(2, E, HF), layers[l + 1]["C"]["w"],
                         layers[l + 1]["C"]["b"].reshape(1, H))

    m = params["mlp"]
    return _readout(hcur, ah, nd,
                    m[0]["w"], m[0]["b"].reshape(1, -1),
                    m[1]["w"], m[1]["b"].reshape(1, -1),
                    m[2]["w"], m[2]["b"].reshape(1, -1))


# speedup vs baseline: 1.7825x; 1.0025x over previous
import jax
import jax.numpy as jnp
from jax import lax
from jax.experimental import pallas as pl
from jax.experimental.pallas import tpu as pltpu
from jax.experimental.pallas import tpu_sc as plsc

N = 10000
E = 320000
H = 128
HF = 64
ATOM_PAD = 32
BOND_PAD = 8

TN = 400
TE = 512

NS = 16
EPAD = 20480
EPR = NS * EPAD
C = 64
NCHUNK = EPAD // C
NP = 10240
NPS = NP // NS
ZR = 32


def _node0_body(hid_ref, emb_ref, w_ref, b_ref, h_ref, ah_ref, db_ref, eh_ref):
    idx = hid_ref[...]
    oh = (idx == lax.broadcasted_iota(jnp.int32, (TN, ATOM_PAD), 1)
          ).astype(jnp.float32)
    h = oh @ emb_ref[...]
    h_ref[...] = h
    p = h @ w_ref[...] + b_ref[...]
    ah_ref[...] = p[:, 0:H]
    db_ref[...] = jnp.stack([p[:, 128:256], p[:, 256:384]])
    eh_ref[...] = jnp.stack(
        [p[:, 384:512],
         jnp.concatenate([p[:, 448:512], p[:, 384:448]], axis=1)])


def _nodeu_body(h_ref, ahp_ref, nd_ref, w_ref, b_ref,
                hn_ref, ah_ref, db_ref, eh_ref):
    nd = nd_ref[...]
    num = jnp.concatenate([nd[0, :, 0:HF], nd[1, :, 0:HF]], axis=1)
    den = jnp.concatenate([nd[0, :, HF:H], nd[1, :, HF:H]], axis=1)
    h = h_ref[...] + jnp.maximum(ahp_ref[...] + num / (den + 1e-6), 0.0)
    hn_ref[...] = h
    p = h @ w_ref[...] + b_ref[...]
    ah_ref[...] = p[:, 0:H]
    db_ref[...] = jnp.stack([p[:, 128:256], p[:, 256:384]])
    eh_ref[...] = jnp.stack(
        [p[:, 384:512],
         jnp.concatenate([p[:, 448:512], p[:, 384:448]], axis=1)])


def _edge0_body(eid_ref, emb_ref, w_ref, b_ref, cee_ref):
    idx = eid_ref[...]
    oh = (idx == lax.broadcasted_iota(jnp.int32, (TE, BOND_PAD), 1)
          ).astype(jnp.float32)
    ee = oh @ emb_ref[...]
    ce = ee @ w_ref[...] + b_ref[...]
    cee_ref[...] = jnp.stack(
        [jnp.concatenate([ce[:, 0:HF], ee[:, 0:HF]], axis=1),
         jnp.concatenate([ce[:, HF:H], ee[:, HF:H]], axis=1)])


def _edgeu_body(eo_ref, w_ref, b_ref, cee_ref):
    eo = eo_ref[...]
    ef = jnp.concatenate([eo[0], eo[1]], axis=1)
    ce = ef @ w_ref[...] + b_ref[...]
    cee_ref[...] = jnp.stack(
        [jnp.concatenate([ce[:, 0:HF], eo[0]], axis=1),
         jnp.concatenate([ce[:, HF:H], eo[1]], axis=1)])


def _readout_body(h_ref, ahp_ref, nd_ref, w0, b0, w1, b1, w2, b2, o_ref):
    nd = nd_ref[...]
    num = jnp.concatenate([nd[0, 0:N, 0:HF], nd[1, 0:N, 0:HF]], axis=1)
    den = jnp.concatenate([nd[0, 0:N, HF:H], nd[1, 0:N, HF:H]], axis=1)
    h = h_ref[...] + jnp.maximum(ahp_ref[...] + num / (den + 1e-6), 0.0)
    s = jnp.sum(h, axis=0, keepdims=True)
    m = jnp.max(h, axis=0, keepdims=True)
    hg = jnp.concatenate([s, m, s / N], axis=1)
    x = jnp.maximum(hg @ w0[...] + b0[...], 0.0)
    x = jnp.maximum(x @ w1[...] + b1[...], 0.0)
    o_ref[...] = x @ w2[...] + b2[...]


def _rep(shape):
    return pl.BlockSpec(shape, lambda i: tuple(0 for _ in shape))


_node_out_specs = [
    pl.BlockSpec((TN, H), lambda i: (i, 0)),
    pl.BlockSpec((TN, H), lambda i: (i, 0)),
    pl.BlockSpec((2, TN, H), lambda i: (0, i, 0)),
    pl.BlockSpec((2, TN, H), lambda i: (0, i, 0)),
]
_node_out_shape = [
    jax.ShapeDtypeStruct((N, H), jnp.float32),
    jax.ShapeDtypeStruct((N, H), jnp.float32),
    jax.ShapeDtypeStruct((2, N, H), jnp.float32),
    jax.ShapeDtypeStruct((2, NP, H), jnp.float32),
]

_node0 = pl.pallas_call(
    _node0_body,
    grid=(N // TN,),
    in_specs=[
        pl.BlockSpec((TN, 1), lambda i: (i, 0)),
        _rep((ATOM_PAD, H)),
        _rep((H, 512)),
        _rep((1, 512)),
    ],
    out_specs=_node_out_specs,
    out_shape=_node_out_shape,
)

_nodeu = pl.pallas_call(
    _nodeu_body,
    grid=(N // TN,),
    in_specs=[
        pl.BlockSpec((TN, H), lambda i: (i, 0)),
        pl.BlockSpec((TN, H), lambda i: (i, 0)),
        pl.BlockSpec((2, TN, H), lambda i: (0, i, 0)),
        _rep((H, 512)),
        _rep((1, 512)),
    ],
    out_specs=_node_out_specs,
    out_shape=_node_out_shape,
)

_edge0 = pl.pallas_call(
    _edge0_body,
    grid=(E // TE,),
    in_specs=[
        pl.BlockSpec((TE, 1), lambda i: (i, 0)),
        _rep((BOND_PAD, H)),
        _rep((H, H)),
        _rep((1, H)),
    ],
    out_specs=pl.BlockSpec((2, TE, H), lambda i: (0, i, 0)),
    out_shape=jax.ShapeDtypeStruct((2, EPR, H), jnp.float32),
)

_edgeu = pl.pallas_call(
    _edgeu_body,
    grid=(E // TE,),
    in_specs=[
        pl.BlockSpec((2, TE, HF), lambda i: (0, i, 0)),
        _rep((H, H)),
        _rep((1, H)),
    ],
    out_specs=pl.BlockSpec((2, TE, H), lambda i: (0, i, 0)),
    out_shape=jax.ShapeDtypeStruct((2, EPR, H), jnp.float32),
)

_readout = pl.pallas_call(
    _readout_body,
    out_shape=jax.ShapeDtypeStruct((1, 1), jnp.float32),
)


def _sc_body(src_hbm, dst_hbm, cee_hbm, db_hbm, ehh_hbm,
             eout_hbm, nd_hbm,
             acc, src_v, dst_v, sadj_v, dadj_v, cee_v, dbr_v, ehr_v,
             stage_v, eo_v, zero_v, sem1, sem2):
    c = lax.axis_index("c")
    s = lax.axis_index("s")
    cn = c * N

    def zrow(j, _):
        for k in range(H // 16):
            zero_v[j, pl.ds(k * 16, 16)] = jnp.zeros((16,), jnp.float32)
        return _
    lax.fori_loop(0, ZR, zrow, 0)
    for i in range(NPS // ZR):
        pltpu.sync_copy(zero_v, acc.at[pl.ds(s * NPS + i * ZR, ZR)])
    plsc.subcore_barrier()

    def edge(j, _2):
        for k in range(HF // 16):
            sl = pl.ds(k * 16, 16)
            sl2 = pl.ds(HF + k * 16, 16)
            ehat = cee_v[j, sl] + dbr_v[j, sl] + ehr_v[j, sl]
            sig = 1.0 / (1.0 + jnp.exp(-ehat))
            stage_v[j, sl] = sig * dbr_v[j, sl2]
            stage_v[j, sl2] = sig
            eo_v[j, sl] = cee_v[j, sl2] + jnp.maximum(ehat, 0.0)
        return _2

    cnp = c * NP

    def chunk(g, carry):
        base = s * EPAD + g * C
        pltpu.sync_copy(src_hbm.at[pl.ds(base, C)], src_v)
        pltpu.sync_copy(dst_hbm.at[pl.ds(base, C)], dst_v)
        for k in range(C // 16):
            sl = pl.ds(k * 16, 16)
            sadj_v[sl] = src_v[sl] + cn
            dadj_v[sl] = dst_v[sl] + cnp
        g1 = pltpu.async_copy(db_hbm.at[sadj_v], dbr_v, sem1)
        g2 = pltpu.async_copy(ehh_hbm.at[dadj_v], ehr_v, sem2)
        ebase = c * EPR + base
        pltpu.sync_copy(cee_hbm.at[pl.ds(ebase, C)], cee_v)
        g1.wait()
        g2.wait()

        lax.fori_loop(0, C, edge, 0)

        pltpu.sync_copy(eo_v, eout_hbm.at[pl.ds(ebase, C)])
        pltpu.sync_copy(stage_v, acc.at[dst_v], add=True)
        return carry
    lax.fori_loop(0, NCHUNK, chunk, 0)

    plsc.subcore_barrier()
    for i in range(NPS // ZR):
        pltpu.sync_copy(acc.at[pl.ds(s * NPS + i * ZR, ZR)], zero_v)
        pltpu.sync_copy(zero_v, nd_hbm.at[pl.ds(c * NP + s * NPS + i * ZR, ZR)])


_sc_layer = pl.kernel(
    _sc_body,
    mesh=plsc.VectorSubcoreMesh(core_axis_name="c", subcore_axis_name="s"),
    out_type=[
        jax.ShapeDtypeStruct((2 * EPR, HF), jnp.float32),
        jax.ShapeDtypeStruct((2 * NP, H), jnp.float32),
    ],
    scratch_types=[
        pltpu.VMEM_SHARED((NP, H), jnp.float32),
        pltpu.VMEM((C,), jnp.int32),
        pltpu.VMEM((C,), jnp.int32),
        pltpu.VMEM((C,), jnp.int32),
        pltpu.VMEM((C,), jnp.int32),
        pltpu.VMEM((C, H), jnp.float32),
        pltpu.VMEM((C, H), jnp.float32),
        pltpu.VMEM((C, H), jnp.float32),
        pltpu.VMEM((C, H), jnp.float32),
        pltpu.VMEM((C, HF), jnp.float32),
        pltpu.VMEM((ZR, H), jnp.float32),
        pltpu.SemaphoreType.DMA,
        pltpu.SemaphoreType.DMA,
    ],
)


def _pack_node_w(layer):
    wa, wb = layer["A"]["w"], layer["B"]["w"]
    wd, we = layer["D"]["w"], layer["Eh"]["w"]
    w = jnp.concatenate([wa, wd[:, :HF], wb[:, :HF], wd[:, HF:], wb[:, HF:],
                         we], axis=1)
    ba, bb = layer["A"]["b"], layer["B"]["b"]
    bd, be = layer["D"]["b"], layer["Eh"]["b"]
    b = jnp.concatenate([ba, bd[:HF], bb[:HF], bd[HF:], bb[HF:], be])
    return w, b.reshape(1, 512)


def kernel(h, e, edge_index, params):
    layers = params["layers"]
    emb_h = jnp.concatenate(
        [params["emb_h"], jnp.zeros((ATOM_PAD - params["emb_h"].shape[0], H),
                                    jnp.float32)], axis=0)
    emb_e = jnp.concatenate(
        [params["emb_e"], jnp.zeros((BOND_PAD - params["emb_e"].shape[0], H),
                                    jnp.float32)], axis=0)
    src = jnp.concatenate(
        [edge_index[0].astype(jnp.int32), jnp.zeros((EPR - E,), jnp.int32)])
    dst = jnp.concatenate(
        [edge_index[1].astype(jnp.int32),
         jnp.full((EPR - E,), N, jnp.int32)])
    hid = h.astype(jnp.int32).reshape(N, 1)
    eid = e.astype(jnp.int32).reshape(E, 1)

    w0, b0 = _pack_node_w(layers[0])
    hcur, ah, db, ehf = _node0(hid, emb_h, w0, b0)
    cee = _edge0(eid, emb_e, layers[0]["C"]["w"],
                 layers[0]["C"]["b"].reshape(1, H))

    nd = None
    for l in range(4):
        eout, nd = _sc_layer(src, dst, cee.reshape(2 * EPR, H),
                             db.reshape(2 * N, H), ehf.reshape(2 * NP, H))
        nd = nd.reshape(2, NP, H)
        if l < 3:
            wl, bl = _pack_node_w(layers[l + 1])
            hcur, ah, db, ehf = _nodeu(hcur, ah, nd, wl, bl)
            cee = _edgeu(eout.reshape(2, EPR, HF), layers[l + 1]["C"]["w"],
                         layers[l + 1]["C"]["b"].reshape(1, H))

    m = params["mlp"]
    return _readout(hcur, ah, nd,
                    m[0]["w"], m[0]["b"].reshape(1, -1),
                    m[1]["w"], m[1]["b"].reshape(1, -1),
                    m[2]["w"], m[2]["b"].reshape(1, -1))
